# Initial kernel scaffold; baseline (speedup 1.0000x reference)
import functools

import jax
import jax.numpy as jnp
from jax import lax
from jax.experimental import pallas as pl
from jax.experimental.pallas import tpu as pltpu
from jax.experimental.pallas import tpu_sc as plsc

NC = 2
NS = 16
NW = NC * NS

CH = 128
NBUF = 4

EMB_D = 64


def _build(n_rows):
    n_chunks = n_rows // CH
    cpw = n_chunks // NW

    mesh = plsc.VectorSubcoreMesh(
        core_axis_name="c", subcore_axis_name="s",
        num_cores=NC, num_subcores=NS,
    )

    @functools.partial(
        pl.kernel,
        out_type=jax.ShapeDtypeStruct((n_rows, EMB_D), jnp.float32),
        mesh=mesh,
        scratch_types=[
            pltpu.VMEM((cpw, CH), jnp.int32),
            pltpu.VMEM((NBUF, CH, EMB_D), jnp.float32),
            pltpu.SemaphoreType.DMA((NBUF,)),
        ],
    )
    def k(table_hbm, idx_hbm, out_hbm, idx_v, buf, gsem):
        wid = lax.axis_index("s") * NC + lax.axis_index("c")
        c0 = wid * cpw

        pltpu.sync_copy(idx_hbm.at[pl.ds(c0, cpw)], idx_v)

        def start_gather(c_local, b):
            pltpu.async_copy(
                table_hbm.at[idx_v.at[c_local]], buf.at[b], gsem.at[b]
            ).start()

        def finish(c_local, b):
            pltpu.make_async_copy(
                table_hbm.at[idx_v.at[c_local]], buf.at[b], gsem.at[b]
            ).wait()
            pltpu.sync_copy(
                buf.at[b], out_hbm.at[pl.ds((c0 + c_local) * CH, CH)]
            )

        for b in range(NBUF):
            start_gather(b, b)

        def body(i, _):
            go = i * NBUF
            for b in range(NBUF):
                finish(go + b, b)
                start_gather(go + b + NBUF, b)
            return 0

        lax.fori_loop(0, cpw // NBUF - 1, body, 0)

        tail = cpw - NBUF
        for b in range(NBUF):
            finish(tail + b, b)

    return k


def kernel(idxs, table):
    b, s = idxs.shape
    d = table.shape[1]
    flat = idxs.reshape(-1).astype(jnp.int32)
    idx2d = flat.reshape(flat.shape[0] // CH, CH)
    out = _build(flat.shape[0])(table, idx2d)
    return out.reshape(b, s, d)


# speedup vs baseline: 1.8765x; 1.8765x over previous
import functools

import jax
import jax.numpy as jnp
from jax import lax
from jax.experimental import pallas as pl
from jax.experimental.pallas import tpu as pltpu
from jax.experimental.pallas import tpu_sc as plsc

NC = 2
NS = 16
NW = NC * NS

CH = 128
NBUF = 4

EMB_D = 64


def _build(n_rows):
    n_chunks = n_rows // CH
    cpw = n_chunks // NW

    mesh = plsc.VectorSubcoreMesh(
        core_axis_name="c", subcore_axis_name="s",
        num_cores=NC, num_subcores=NS,
    )

    @functools.partial(
        pl.kernel,
        out_type=jax.ShapeDtypeStruct((n_rows, EMB_D), jnp.float32),
        mesh=mesh,
        scratch_types=[
            pltpu.VMEM((cpw, CH), jnp.int32),
            pltpu.VMEM((NBUF, CH, EMB_D), jnp.float32),
            pltpu.SemaphoreType.DMA((NBUF,)),
        ],
        compiler_params=pltpu.CompilerParams(use_tc_tiling_on_sc=False),
    )
    def k(table_hbm, idx_hbm, out_hbm, idx_v, buf, gsem):
        wid = lax.axis_index("s") * NC + lax.axis_index("c")
        c0 = wid * cpw

        pltpu.sync_copy(idx_hbm.at[pl.ds(c0, cpw)], idx_v)

        def start_gather(c_local, b):
            pltpu.async_copy(
                table_hbm.at[idx_v.at[c_local]], buf.at[b], gsem.at[b]
            )

        def finish(c_local, b):
            pltpu.make_async_copy(
                table_hbm.at[idx_v.at[c_local]], buf.at[b], gsem.at[b]
            ).wait()
            pltpu.sync_copy(
                buf.at[b], out_hbm.at[pl.ds((c0 + c_local) * CH, CH)]
            )

        for b in range(NBUF):
            start_gather(b, b)

        def body(i, _):
            go = i * NBUF
            for b in range(NBUF):
                finish(go + b, b)
                start_gather(go + b + NBUF, b)
            return 0

        lax.fori_loop(0, cpw // NBUF - 1, body, 0)

        tail = cpw - NBUF
        for b in range(NBUF):
            finish(tail + b, b)

    return k


def kernel(idxs, table):
    b, s = idxs.shape
    d = table.shape[1]
    flat = idxs.reshape(-1).astype(jnp.int32)
    idx2d = flat.reshape(flat.shape[0] // CH, CH)
    out = _build(flat.shape[0])(table, idx2d)
    return out.reshape(b, s, d)
